# baseline (speedup 1.0000x reference)
import functools

import jax
import jax.numpy as jnp
from jax import lax
from jax.experimental import pallas as pl
from jax.experimental.pallas import tpu as pltpu, tpu_sc as plsc

EXPERTS = 64
LANES = 16
NUM_WORKERS = 16
ACC = 2 * EXPERTS * LANES


@functools.lru_cache(maxsize=None)
def _build(n_total: int):
    per_w = n_total // NUM_WORKERS
    n_vecs = per_w // LANES
    scale = float(EXPERTS) / (float(n_total) * float(n_total))
    gsum_off = EXPERTS * LANES

    mesh = plsc.VectorSubcoreMesh(
        core_axis_name="c", subcore_axis_name="s", num_cores=1,
        num_subcores=NUM_WORKERS,
    )

    @functools.partial(
        pl.kernel,
        out_type=jax.ShapeDtypeStruct((LANES,), jnp.float32),
        mesh=mesh,
        compiler_params=pltpu.CompilerParams(
            needs_layout_passes=False, skip_device_barrier=True
        ),
        scratch_types=[
            pltpu.VMEM((per_w,), jnp.int32),
            pltpu.VMEM((per_w,), jnp.float32),
            pltpu.VMEM((ACC,), jnp.float32),
            pltpu.VMEM((ACC,), jnp.float32),
            pltpu.VMEM((LANES,), jnp.float32),
            pltpu.VMEM_SHARED((NUM_WORKERS, ACC), jnp.float32),
        ],
    )
    def lbloss(idx_hbm, gat_hbm, out_hbm,
               idx_v, gat_v, acc_v, tmp_v, res_v, sh_all):
        sid = lax.axis_index("s")
        base = sid * per_w

        pltpu.sync_copy(idx_hbm.at[pl.ds(base, per_w)], idx_v)
        pltpu.sync_copy(gat_hbm.at[pl.ds(base, per_w)], gat_v)

        lane = lax.iota(jnp.int32, LANES)
        zeros = jnp.zeros((LANES,), jnp.float32)
        ones = jnp.ones((LANES,), jnp.float32)

        def zero_body(j, carry):
            acc_v[pl.ds(j * LANES, LANES)] = zeros
            return carry

        lax.fori_loop(0, ACC // LANES, zero_body, 0)

        def acc_body(i, carry):
            iv = idx_v[pl.ds(i * LANES, LANES)]
            gv = gat_v[pl.ds(i * LANES, LANES)]
            addr = iv * LANES + lane
            plsc.addupdate_scatter(acc_v, [addr], ones)
            plsc.addupdate_scatter(acc_v, [addr + gsum_off], gv)
            return carry

        lax.fori_loop(0, n_vecs, acc_body, 0)

        pltpu.sync_copy(acc_v, sh_all.at[sid])
        plsc.subcore_barrier()

        @pl.when(sid == 0)
        def _finalize():
            def red_body(w, carry):
                pltpu.sync_copy(sh_all.at[w], tmp_v)
                for j in range(ACC // LANES):
                    sl = pl.ds(j * LANES, LANES)
                    acc_v[sl] = acc_v[sl] + tmp_v[sl]
                return carry

            lax.fori_loop(1, NUM_WORKERS, red_body, 0)

            def dot_body(e, acc):
                hs = jnp.sum(acc_v[pl.ds(e * LANES, LANES)])
                gs = jnp.sum(acc_v[pl.ds(gsum_off + e * LANES, LANES)])
                return acc + hs * gs

            acc = lax.fori_loop(0, EXPERTS, dot_body, jnp.float32(0.0))
            res_v[...] = ones * (acc * jnp.float32(scale))
            pltpu.sync_copy(res_v, out_hbm)

    return lbloss


def kernel(gates, indices):
    n = gates.size
    flat_g = gates.reshape(n)
    flat_i = indices.reshape(n).astype(jnp.int32)
    out = _build(n)(flat_i, flat_g)
    return out[0]


# speedup vs baseline: 2.4712x; 2.4712x over previous
import functools

import jax
import jax.numpy as jnp
from jax import lax
from jax.experimental import pallas as pl
from jax.experimental.pallas import tpu as pltpu, tpu_sc as plsc

EXPERTS = 64
LANES = 16
NUM_WORKERS = 16
ACC = 2 * EXPERTS * LANES


@functools.lru_cache(maxsize=None)
def _build(batch: int, top_k: int, seq: int):
    n_total = batch * seq * top_k
    per_w = n_total // NUM_WORKERS
    n_rows = batch * top_k
    w_per_row = NUM_WORKERS // n_rows
    n_vecs = per_w // LANES
    scale = float(EXPERTS) / (float(n_total) * float(n_total))
    gsum_off = EXPERTS * LANES

    mesh = plsc.VectorSubcoreMesh(
        core_axis_name="c", subcore_axis_name="s", num_cores=1,
        num_subcores=NUM_WORKERS,
    )

    @functools.partial(
        pl.kernel,
        out_type=jax.ShapeDtypeStruct((LANES,), jnp.float32),
        mesh=mesh,
        compiler_params=pltpu.CompilerParams(
            needs_layout_passes=False, use_tc_tiling_on_sc=False
        ),
        scratch_types=[
            pltpu.VMEM((per_w,), jnp.int32),
            pltpu.VMEM((per_w,), jnp.float32),
            pltpu.VMEM((ACC,), jnp.float32),
            pltpu.VMEM((ACC,), jnp.float32),
            pltpu.VMEM((LANES,), jnp.float32),
            pltpu.VMEM_SHARED((NUM_WORKERS, ACC), jnp.float32),
            pltpu.SemaphoreType.DMA,
            pltpu.SemaphoreType.DMA,
        ],
    )
    def lbloss(gat_hbm, idx_hbm, out_hbm,
               idx_v, gat_v, acc_v, tmp_v, res_v, sh_all, sem_i, sem_g):
        sid = lax.axis_index("s")
        row = sid // w_per_row
        b = row // top_k
        k = row % top_k
        s0 = (sid % w_per_row) * per_w

        cp_i = pltpu.async_copy(idx_hbm.at[b, k, pl.ds(s0, per_w)], idx_v, sem_i)
        cp_g = pltpu.async_copy(gat_hbm.at[b, k, pl.ds(s0, per_w)], gat_v, sem_g)

        lane = lax.iota(jnp.int32, LANES)
        zeros = jnp.zeros((LANES,), jnp.float32)
        ones = jnp.ones((LANES,), jnp.float32)

        def zero_body(j, carry):
            acc_v[pl.ds(j * LANES, LANES)] = zeros
            return carry

        lax.fori_loop(0, ACC // LANES, zero_body, 0)

        cp_i.wait()
        cp_g.wait()

        def acc_body(i, carry):
            iv = idx_v[pl.ds(i * LANES, LANES)]
            gv = gat_v[pl.ds(i * LANES, LANES)]
            addr = iv * LANES + lane
            plsc.addupdate_scatter(acc_v, [addr], ones)
            plsc.addupdate_scatter(acc_v, [addr + gsum_off], gv)
            return carry

        lax.fori_loop(0, n_vecs, acc_body, 0)

        pltpu.sync_copy(acc_v, sh_all.at[sid])
        plsc.subcore_barrier()

        @pl.when(sid == 0)
        def _finalize():
            def red_body(w, carry):
                pltpu.sync_copy(sh_all.at[w], tmp_v)
                for j in range(ACC // LANES):
                    sl = pl.ds(j * LANES, LANES)
                    acc_v[sl] = acc_v[sl] + tmp_v[sl]
                return carry

            lax.fori_loop(1, NUM_WORKERS, red_body, 0)

            def dot_body(e, acc):
                hs = jnp.sum(acc_v[pl.ds(e * LANES, LANES)])
                gs = jnp.sum(acc_v[pl.ds(gsum_off + e * LANES, LANES)])
                return acc + hs * gs

            acc = lax.fori_loop(0, EXPERTS, dot_body, jnp.float32(0.0))
            res_v[...] = ones * (acc * jnp.float32(scale))
            pltpu.sync_copy(res_v, out_hbm)

    return lbloss


def kernel(gates, indices):
    b, s, k = gates.shape
    gt = jnp.transpose(gates, (0, 2, 1))
    it = jnp.transpose(indices.astype(jnp.int32), (0, 2, 1))
    out = _build(b, k, s)(gt, it)
    return out[0]
